# baseline (speedup 1.0000x reference)
import functools

import jax
import jax.numpy as jnp
from jax import lax
from jax.experimental import pallas as pl
from jax.experimental.pallas import tpu as pltpu
from jax.experimental.pallas import tpu_sc as plsc

N = 10000
D = 128
E = 320000
NC = 2
NS = 16
NW = NC * NS
CH = 128
NB = 2
NCH = ((-(-E // (NW * CH)) + NB - 1) // NB) * NB
EPW = NCH * CH
EPAD = EPW * NW
NPAD = 10240
RPW = NPAD // NS

_mesh = plsc.VectorSubcoreMesh(core_axis_name="c", subcore_axis_name="s")


@functools.partial(
    pl.kernel,
    out_type=jax.ShapeDtypeStruct((NW, NPAD), jnp.float32),
    mesh=_mesh,
    compiler_params=pltpu.CompilerParams(needs_layout_passes=False),
    scratch_types=[
        pltpu.VMEM((NCH, CH), jnp.int32),
        pltpu.VMEM((NPAD,), jnp.float32),
    ],
)
def _sc_degree(dst_hbm, z1_hbm, out_hbm, dst_v, deg_v):
    c = lax.axis_index("c")
    s = lax.axis_index("s")
    wid = c * NS + s
    pltpu.sync_copy(z1_hbm, deg_v)
    pltpu.sync_copy(dst_hbm.at[wid], dst_v)
    ones = jnp.ones((16,), jnp.float32)

    def body(j, carry):
        for k in range(CH // 16):
            idx = dst_v[j, pl.ds(k * 16, 16)]
            plsc.addupdate_scatter(deg_v, [idx], ones)
        return carry

    lax.fori_loop(0, NCH, body, 0)
    pltpu.sync_copy(deg_v, out_hbm.at[wid])


def _tc_prescale_body(x_ref, w1_ref, degp_ref, p_ref, dinv8_ref):
    deg = jnp.sum(degp_ref[...], axis=0) + 1.0
    dinv = lax.rsqrt(deg)
    h = jnp.dot(x_ref[...], w1_ref[...], preferred_element_type=jnp.float32)
    dcol = dinv[:, None]
    p_ref[...] = dcol * h
    dinv8_ref[...] = jnp.broadcast_to(dcol, dinv8_ref.shape)


def _tc_prescale(x_pad, W1, deg_parts):
    blk = 1280
    grid = NPAD // blk
    return pl.pallas_call(
        _tc_prescale_body,
        grid=(grid,),
        in_specs=[
            pl.BlockSpec((blk, D), lambda i: (i, 0)),
            pl.BlockSpec((D, D), lambda i: (0, 0)),
            pl.BlockSpec((NW, blk), lambda i: (0, i)),
        ],
        out_specs=[
            pl.BlockSpec((blk, D), lambda i: (i, 0)),
            pl.BlockSpec((blk, 8), lambda i: (i, 0)),
        ],
        out_shape=[
            jax.ShapeDtypeStruct((NPAD, D), jnp.float32),
            jax.ShapeDtypeStruct((NPAD, 8), jnp.float32),
        ],
    )(x_pad, W1, deg_parts)


@functools.partial(
    pl.kernel,
    out_type=jax.ShapeDtypeStruct((NC, NPAD, D), jnp.float32),
    mesh=_mesh,
    scratch_types=[
        pltpu.VMEM((NCH, CH), jnp.int32),
        pltpu.VMEM((NCH, CH), jnp.int32),
        pltpu.VMEM((CH, D), jnp.float32),
        pltpu.VMEM_SHARED((NPAD, D), jnp.float32),
        pltpu.SemaphoreType.DMA,
    ],
)
def _sc_edge_pass(p_hbm, src_hbm, dst_hbm, z2d_hbm, s_out,
                  src_v, dst_v, rows_v, s_sh, sem0):
    c = lax.axis_index("c")
    s = lax.axis_index("s")
    wid = c * NS + s
    pltpu.sync_copy(z2d_hbm, s_sh.at[pl.ds(s * RPW, RPW)])
    pltpu.sync_copy(src_hbm.at[wid], src_v)
    pltpu.sync_copy(dst_hbm.at[wid], dst_v)
    plsc.subcore_barrier()

    def body(j, carry):
        pltpu.async_copy(p_hbm.at[src_v.at[j]], rows_v, sem0).wait()
        pltpu.sync_copy(rows_v, s_sh.at[dst_v.at[j]], add=True)
        return carry

    lax.fori_loop(0, NCH, body, 0)
    plsc.subcore_barrier()
    pltpu.sync_copy(s_sh.at[pl.ds(s * RPW, RPW)],
                    s_out.at[c, pl.ds(s * RPW, RPW)])


@functools.partial(
    pl.kernel,
    out_type=jax.ShapeDtypeStruct((NW, NPAD), jnp.float32),
    mesh=_mesh,
    compiler_params=pltpu.CompilerParams(needs_layout_passes=False),
    scratch_types=[
        pltpu.VMEM((NCH, CH), jnp.int32),
        pltpu.VMEM((NCH, CH), jnp.int32),
        pltpu.VMEM((NPAD,), jnp.float32),
        pltpu.VMEM((NPAD,), jnp.float32),
    ],
)
def _sc_t_pass(dinv_hbm, src_hbm, dst_hbm, z1_hbm, out_hbm,
               src_v, dst_v, dinv_v, t_v):
    c = lax.axis_index("c")
    s = lax.axis_index("s")
    wid = c * NS + s
    pltpu.sync_copy(z1_hbm, t_v)
    pltpu.sync_copy(dinv_hbm, dinv_v)
    pltpu.sync_copy(src_hbm.at[wid], src_v)
    pltpu.sync_copy(dst_hbm.at[wid], dst_v)

    def body(j, carry):
        for k in range(CH // 16):
            d16 = dst_v[j, pl.ds(k * 16, 16)]
            s16 = src_v[j, pl.ds(k * 16, 16)]
            vals = plsc.load_gather(dinv_v, [d16])
            plsc.addupdate_scatter(t_v, [s16], vals)
        return carry

    lax.fori_loop(0, NCH, body, 0)
    pltpu.sync_copy(t_v, out_hbm.at[wid])


def _tc_finish_body(sp_ref, p_ref, dinv8_ref, tp_ref, w2_ref, b1_ref,
                    b2_ref, wfc_ref, bfc_ref, out_ref, acc_ref):
    i = pl.program_id(0)
    blk = sp_ref.shape[1]

    @pl.when(i == 0)
    def _init():
        acc_ref[...] = jnp.zeros_like(acc_ref)

    dcol = dinv8_ref[:, 0:1]
    S = sp_ref[0] + sp_ref[1] + p_ref[...]
    h1 = jnp.maximum(dcol * S + b1_ref[...], 0.0)
    t = jnp.sum(tp_ref[...], axis=0)[:, None] + dcol
    row = i * blk + lax.broadcasted_iota(jnp.int32, (blk, 1), 0)
    cw = jnp.where(row < N, dcol * t, 0.0)
    acc_ref[...] += jnp.sum(cw * h1, axis=0, keepdims=True)

    @pl.when(i == pl.num_programs(0) - 1)
    def _fin():
        v = acc_ref[...]
        g = jnp.dot(v, w2_ref[...], preferred_element_type=jnp.float32)
        g = g / float(N) + b2_ref[...]
        out_ref[...] = jnp.dot(g, wfc_ref[...],
                               preferred_element_type=jnp.float32) + bfc_ref[...]


def _tc_finish(s_parts, p, dinv8, t_parts, W2, b1, b2, Wfc, bfc):
    blk = 1280
    grid = NPAD // blk
    return pl.pallas_call(
        _tc_finish_body,
        grid=(grid,),
        in_specs=[
            pl.BlockSpec((NC, blk, D), lambda i: (0, i, 0)),
            pl.BlockSpec((blk, D), lambda i: (i, 0)),
            pl.BlockSpec((blk, 8), lambda i: (i, 0)),
            pl.BlockSpec((NW, blk), lambda i: (0, i)),
            pl.BlockSpec((D, D), lambda i: (0, 0)),
            pl.BlockSpec((1, D), lambda i: (0, 0)),
            pl.BlockSpec((1, D), lambda i: (0, 0)),
            pl.BlockSpec((D, 1024), lambda i: (0, 0)),
            pl.BlockSpec((1, 1024), lambda i: (0, 0)),
        ],
        out_specs=pl.BlockSpec((1, 1024), lambda i: (0, 0)),
        out_shape=jax.ShapeDtypeStruct((1, 1024), jnp.float32),
        scratch_shapes=[pltpu.VMEM((1, D), jnp.float32)],
    )(s_parts, p, dinv8, t_parts, W2, b1.reshape(1, D), b2.reshape(1, D),
      Wfc, bfc.reshape(1, 1024))


def kernel(x, edge_index, W1, b1, W2, b2, Wfc, bfc):
    src = edge_index[0].astype(jnp.int32)
    dst = edge_index[1].astype(jnp.int32)
    src_p = jnp.full((EPAD,), N, jnp.int32).at[:E].set(src).reshape(NW, NCH, CH)
    dst_p = jnp.full((EPAD,), N, jnp.int32).at[:E].set(dst).reshape(NW, NCH, CH)
    x_pad = jnp.zeros((NPAD, D), jnp.float32).at[:N].set(x)

    z1 = jnp.zeros((NPAD,), jnp.float32)
    z2d = jnp.zeros((RPW, D), jnp.float32)

    deg_parts = _sc_degree(dst_p, z1)
    p, dinv8 = _tc_prescale(x_pad, W1, deg_parts)
    s_parts = _sc_edge_pass(p, src_p, dst_p, z2d)
    t_parts = _sc_t_pass(dinv8[:, 0], src_p, dst_p, z1)
    out = _tc_finish(s_parts, p, dinv8, t_parts, W2, b1, b2, Wfc, bfc)
    return out.reshape(16, 64)


# speedup vs baseline: 2.0424x; 2.0424x over previous
import functools

import jax
import jax.numpy as jnp
from jax import lax
from jax.experimental import pallas as pl
from jax.experimental.pallas import tpu as pltpu
from jax.experimental.pallas import tpu_sc as plsc

N = 10000
D = 128
E = 320000
NC = 2
NS = 16
NW = NC * NS
CH = 128
NCH = -(-E // (NW * CH))
EPW = NCH * CH
EPAD = EPW * NW
NPAD = 10240
RPW = NPAD // NS

_mesh = plsc.VectorSubcoreMesh(core_axis_name="c", subcore_axis_name="s")


@functools.partial(
    pl.kernel,
    out_type=jax.ShapeDtypeStruct((NW, NPAD), jnp.float32),
    mesh=_mesh,
    compiler_params=pltpu.CompilerParams(needs_layout_passes=False),
    scratch_types=[
        pltpu.VMEM((NCH, CH), jnp.int32),
        pltpu.VMEM((NPAD,), jnp.float32),
    ],
)
def _sc_degree(dst_hbm, z1_hbm, out_hbm, dst_v, deg_v):
    c = lax.axis_index("c")
    s = lax.axis_index("s")
    wid = c * NS + s
    pltpu.sync_copy(z1_hbm, deg_v)
    pltpu.sync_copy(dst_hbm.at[wid], dst_v)
    ones = jnp.ones((16,), jnp.float32)

    def body(j, carry):
        for k in range(CH // 16):
            idx = dst_v[j, pl.ds(k * 16, 16)]
            plsc.addupdate_scatter(deg_v, [idx], ones)
        return carry

    lax.fori_loop(0, NCH, body, 0)
    pltpu.sync_copy(deg_v, out_hbm.at[wid])


def _tc_prescale_body(x_ref, w1_ref, degp_ref, p_ref, dinv8_ref):
    deg = jnp.sum(degp_ref[...], axis=0) + 1.0
    dinv = lax.rsqrt(deg)
    h = jnp.dot(x_ref[...], w1_ref[...], preferred_element_type=jnp.float32)
    dcol = dinv[:, None]
    p_ref[...] = dcol * h
    dinv8_ref[...] = jnp.broadcast_to(dcol, dinv8_ref.shape)


def _tc_prescale(x_pad, W1, deg_parts):
    blk = 1280
    grid = NPAD // blk
    return pl.pallas_call(
        _tc_prescale_body,
        grid=(grid,),
        in_specs=[
            pl.BlockSpec((blk, D), lambda i: (i, 0)),
            pl.BlockSpec((D, D), lambda i: (0, 0)),
            pl.BlockSpec((NW, blk), lambda i: (0, i)),
        ],
        out_specs=[
            pl.BlockSpec((blk, D), lambda i: (i, 0)),
            pl.BlockSpec((blk, 8), lambda i: (i, 0)),
        ],
        out_shape=[
            jax.ShapeDtypeStruct((NPAD, D), jnp.float32),
            jax.ShapeDtypeStruct((NPAD, 8), jnp.float32),
        ],
    )(x_pad, W1, deg_parts)


@functools.partial(
    pl.kernel,
    out_type=jax.ShapeDtypeStruct((NC, NPAD, D), jnp.float32),
    mesh=_mesh,
    scratch_types=[
        pltpu.VMEM((NCH, CH), jnp.int32),
        pltpu.VMEM((NCH, CH), jnp.int32),
        pltpu.VMEM((CH, D), jnp.float32),
        pltpu.VMEM_SHARED((NPAD, D), jnp.float32),
        pltpu.SemaphoreType.DMA,
    ],
)
def _sc_edge_pass(p_hbm, src_hbm, dst_hbm, z2d_hbm, s_out,
                  src_v, dst_v, rows_v, s_sh, sem0):
    c = lax.axis_index("c")
    s = lax.axis_index("s")
    wid = c * NS + s
    pltpu.sync_copy(z2d_hbm, s_sh.at[pl.ds(s * RPW, RPW)])
    pltpu.sync_copy(src_hbm.at[wid], src_v)
    pltpu.sync_copy(dst_hbm.at[wid], dst_v)
    plsc.subcore_barrier()

    def body(j, carry):
        pltpu.async_copy(p_hbm.at[src_v.at[j]], rows_v, sem0).wait()
        pltpu.sync_copy(rows_v, s_sh.at[dst_v.at[j]], add=True)
        return carry

    lax.fori_loop(0, NCH, body, 0)
    plsc.subcore_barrier()
    pltpu.sync_copy(s_sh.at[pl.ds(s * RPW, RPW)],
                    s_out.at[c, pl.ds(s * RPW, RPW)])


@functools.partial(
    pl.kernel,
    out_type=jax.ShapeDtypeStruct((NW, NPAD), jnp.float32),
    mesh=_mesh,
    compiler_params=pltpu.CompilerParams(needs_layout_passes=False),
    scratch_types=[
        pltpu.VMEM((NCH, CH), jnp.int32),
        pltpu.VMEM((NCH, CH), jnp.int32),
        pltpu.VMEM((NPAD,), jnp.float32),
        pltpu.VMEM((NPAD,), jnp.float32),
    ],
)
def _sc_t_pass(dinv_hbm, src_hbm, dst_hbm, z1_hbm, out_hbm,
               src_v, dst_v, dinv_v, t_v):
    c = lax.axis_index("c")
    s = lax.axis_index("s")
    wid = c * NS + s
    pltpu.sync_copy(z1_hbm, t_v)
    pltpu.sync_copy(dinv_hbm, dinv_v)
    pltpu.sync_copy(src_hbm.at[wid], src_v)
    pltpu.sync_copy(dst_hbm.at[wid], dst_v)

    def body(j, carry):
        for k in range(CH // 16):
            d16 = dst_v[j, pl.ds(k * 16, 16)]
            s16 = src_v[j, pl.ds(k * 16, 16)]
            vals = plsc.load_gather(dinv_v, [d16])
            plsc.addupdate_scatter(t_v, [s16], vals)
        return carry

    lax.fori_loop(0, NCH, body, 0)
    pltpu.sync_copy(t_v, out_hbm.at[wid])


def _tc_finish_body(sp_ref, p_ref, dinv8_ref, tp_ref, w2_ref, b1_ref,
                    b2_ref, wfc_ref, bfc_ref, out_ref, acc_ref):
    i = pl.program_id(0)
    blk = sp_ref.shape[1]

    @pl.when(i == 0)
    def _init():
        acc_ref[...] = jnp.zeros_like(acc_ref)

    dcol = dinv8_ref[:, 0:1]
    S = sp_ref[0] + sp_ref[1] + p_ref[...]
    h1 = jnp.maximum(dcol * S + b1_ref[...], 0.0)
    t = jnp.sum(tp_ref[...], axis=0)[:, None] + dcol
    row = i * blk + lax.broadcasted_iota(jnp.int32, (blk, 1), 0)
    cw = jnp.where(row < N, dcol * t, 0.0)
    acc_ref[...] += jnp.sum(cw * h1, axis=0, keepdims=True)

    @pl.when(i == pl.num_programs(0) - 1)
    def _fin():
        v = acc_ref[...]
        g = jnp.dot(v, w2_ref[...], preferred_element_type=jnp.float32)
        g = g / float(N) + b2_ref[...]
        out_ref[...] = jnp.dot(g, wfc_ref[...],
                               preferred_element_type=jnp.float32) + bfc_ref[...]


def _tc_finish(s_parts, p, dinv8, t_parts, W2, b1, b2, Wfc, bfc):
    blk = 1280
    grid = NPAD // blk
    return pl.pallas_call(
        _tc_finish_body,
        grid=(grid,),
        in_specs=[
            pl.BlockSpec((NC, blk, D), lambda i: (0, i, 0)),
            pl.BlockSpec((blk, D), lambda i: (i, 0)),
            pl.BlockSpec((blk, 8), lambda i: (i, 0)),
            pl.BlockSpec((NW, blk), lambda i: (0, i)),
            pl.BlockSpec((D, D), lambda i: (0, 0)),
            pl.BlockSpec((1, D), lambda i: (0, 0)),
            pl.BlockSpec((1, D), lambda i: (0, 0)),
            pl.BlockSpec((D, 1024), lambda i: (0, 0)),
            pl.BlockSpec((1, 1024), lambda i: (0, 0)),
        ],
        out_specs=pl.BlockSpec((1, 1024), lambda i: (0, 0)),
        out_shape=jax.ShapeDtypeStruct((1, 1024), jnp.float32),
        scratch_shapes=[pltpu.VMEM((1, D), jnp.float32)],
    )(s_parts, p, dinv8, t_parts, W2, b1.reshape(1, D), b2.reshape(1, D),
      Wfc, bfc.reshape(1, 1024))


def kernel(x, edge_index, W1, b1, W2, b2, Wfc, bfc):
    src = edge_index[0].astype(jnp.int32)
    dst = edge_index[1].astype(jnp.int32)
    fill = (jnp.arange(EPAD, dtype=jnp.int32) % (NPAD - N)) + N
    src_p = fill.at[:E].set(src).reshape(NW, NCH, CH)
    dst_p = fill.at[:E].set(dst).reshape(NW, NCH, CH)
    x_pad = jnp.zeros((NPAD, D), jnp.float32).at[:N].set(x)

    z1 = jnp.zeros((NPAD,), jnp.float32)
    z2d = jnp.zeros((RPW, D), jnp.float32)

    deg_parts = _sc_degree(dst_p, z1)
    p, dinv8 = _tc_prescale(x_pad, W1, deg_parts)
    s_parts = _sc_edge_pass(p, src_p, dst_p, z2d)
    t_parts = _sc_t_pass(dinv8[:, 0], src_p, dst_p, z1)
    out = _tc_finish(s_parts, p, dinv8, t_parts, W2, b1, b2, Wfc, bfc)
    return out.reshape(16, 64)


# speedup vs baseline: 2.6780x; 1.3112x over previous
import functools

import jax
import jax.numpy as jnp
from jax import lax
from jax.experimental import pallas as pl
from jax.experimental.pallas import tpu as pltpu
from jax.experimental.pallas import tpu_sc as plsc

N = 10000
D = 128
E = 320000
NC = 2
NS = 16
NW = NC * NS
CH = 128
NCH = 80
EPW = NCH * CH
EPAD = EPW * NW
NPAD = 10240
RPW = NPAD // NS

_mesh = plsc.VectorSubcoreMesh(core_axis_name="c", subcore_axis_name="s")


@functools.partial(
    pl.kernel,
    out_type=jax.ShapeDtypeStruct((NW, NPAD), jnp.float32),
    mesh=_mesh,
    compiler_params=pltpu.CompilerParams(needs_layout_passes=False),
    scratch_types=[
        pltpu.VMEM((NCH, CH), jnp.int32),
        pltpu.VMEM((NPAD,), jnp.float32),
    ],
)
def _sc_degree(dst_hbm, z1_hbm, out_hbm, dst_v, deg_v):
    c = lax.axis_index("c")
    s = lax.axis_index("s")
    wid = c * NS + s
    pltpu.sync_copy(z1_hbm, deg_v)
    pltpu.sync_copy(dst_hbm.at[wid], dst_v)
    ones = jnp.ones((16,), jnp.float32)

    def body(j, carry):
        for k in range(CH // 16):
            idx = dst_v[j, pl.ds(k * 16, 16)]
            plsc.addupdate_scatter(deg_v, [idx], ones)
        return carry

    lax.fori_loop(0, NCH, body, 0)
    pltpu.sync_copy(deg_v, out_hbm.at[wid])


def _tc_prescale_body(x_ref, w1_ref, degp_ref, p_ref, dinv8_ref):
    deg = jnp.sum(degp_ref[...], axis=0) + 1.0
    dinv = lax.rsqrt(deg)
    h = jnp.dot(x_ref[...], w1_ref[...], preferred_element_type=jnp.float32)
    dcol = dinv[:, None]
    p_ref[...] = dcol * h
    dinv8_ref[...] = jnp.broadcast_to(dcol, dinv8_ref.shape)


def _tc_prescale(x_pad, W1, deg_parts):
    blk = 1280
    grid = NPAD // blk
    return pl.pallas_call(
        _tc_prescale_body,
        grid=(grid,),
        in_specs=[
            pl.BlockSpec((blk, D), lambda i: (i, 0)),
            pl.BlockSpec((D, D), lambda i: (0, 0)),
            pl.BlockSpec((NW, blk), lambda i: (0, i)),
        ],
        out_specs=[
            pl.BlockSpec((blk, D), lambda i: (i, 0)),
            pl.BlockSpec((blk, 8), lambda i: (i, 0)),
        ],
        out_shape=[
            jax.ShapeDtypeStruct((NPAD, D), jnp.float32),
            jax.ShapeDtypeStruct((NPAD, 8), jnp.float32),
        ],
    )(x_pad, W1, deg_parts)


@functools.partial(
    pl.kernel,
    out_type=jax.ShapeDtypeStruct((NC, NPAD, D), jnp.float32),
    mesh=_mesh,
    scratch_types=[
        pltpu.VMEM((NCH // 2, CH), jnp.int32),
        pltpu.VMEM((NCH // 2, CH), jnp.int32),
        pltpu.VMEM((CH, D), jnp.float32),
        pltpu.VMEM((CH, D), jnp.float32),
        pltpu.VMEM_SHARED((NPAD, D), jnp.float32),
        pltpu.SemaphoreType.DMA,
        pltpu.SemaphoreType.DMA,
    ],
)
def _sc_edge_pass(p_hbm, src_hbm, dst_hbm, z2d_hbm, s_out,
                  src_v, dst_v, rb0, rb1, s_sh, g0, g1):
    HNCH = NCH // 2
    c = lax.axis_index("c")
    s = lax.axis_index("s")
    wid = c * NS + s
    pltpu.sync_copy(z2d_hbm, s_sh.at[pl.ds(s * RPW, RPW)])
    plsc.subcore_barrier()

    def body(j0, carry):
        j = j0 * 2
        pltpu.async_copy(p_hbm.at[src_v.at[j + 1]], rb1, g1)
        pltpu.make_async_copy(p_hbm.at[src_v.at[j]], rb0, g0).wait()
        pltpu.sync_copy(rb0, s_sh.at[dst_v.at[j]], add=True)
        jn = jnp.minimum(j + 2, HNCH - 1)
        pltpu.async_copy(p_hbm.at[src_v.at[jn]], rb0, g0)
        pltpu.make_async_copy(p_hbm.at[src_v.at[j + 1]], rb1, g1).wait()
        pltpu.sync_copy(rb1, s_sh.at[dst_v.at[j + 1]], add=True)
        return carry

    for h in range(2):
        pltpu.sync_copy(src_hbm.at[wid, pl.ds(h * HNCH, HNCH)], src_v)
        pltpu.sync_copy(dst_hbm.at[wid, pl.ds(h * HNCH, HNCH)], dst_v)
        pltpu.async_copy(p_hbm.at[src_v.at[0]], rb0, g0)
        lax.fori_loop(0, HNCH // 2, body, 0)
        pltpu.make_async_copy(p_hbm.at[src_v.at[HNCH - 1]], rb0, g0).wait()

    plsc.subcore_barrier()
    pltpu.sync_copy(s_sh.at[pl.ds(s * RPW, RPW)],
                    s_out.at[c, pl.ds(s * RPW, RPW)])


@functools.partial(
    pl.kernel,
    out_type=jax.ShapeDtypeStruct((NW, NPAD), jnp.float32),
    mesh=_mesh,
    compiler_params=pltpu.CompilerParams(needs_layout_passes=False),
    scratch_types=[
        pltpu.VMEM((NCH, CH), jnp.int32),
        pltpu.VMEM((NCH, CH), jnp.int32),
        pltpu.VMEM((NPAD,), jnp.float32),
        pltpu.VMEM((NPAD,), jnp.float32),
    ],
)
def _sc_t_pass(dinv_hbm, src_hbm, dst_hbm, z1_hbm, out_hbm,
               src_v, dst_v, dinv_v, t_v):
    c = lax.axis_index("c")
    s = lax.axis_index("s")
    wid = c * NS + s
    pltpu.sync_copy(z1_hbm, t_v)
    pltpu.sync_copy(dinv_hbm, dinv_v)
    pltpu.sync_copy(src_hbm.at[wid], src_v)
    pltpu.sync_copy(dst_hbm.at[wid], dst_v)

    def body(j, carry):
        for k in range(CH // 16):
            d16 = dst_v[j, pl.ds(k * 16, 16)]
            s16 = src_v[j, pl.ds(k * 16, 16)]
            vals = plsc.load_gather(dinv_v, [d16])
            plsc.addupdate_scatter(t_v, [s16], vals)
        return carry

    lax.fori_loop(0, NCH, body, 0)
    pltpu.sync_copy(t_v, out_hbm.at[wid])


def _tc_finish_body(sp_ref, p_ref, dinv8_ref, tp_ref, w2_ref, b1_ref,
                    b2_ref, wfc_ref, bfc_ref, out_ref, acc_ref):
    i = pl.program_id(0)
    blk = sp_ref.shape[1]

    @pl.when(i == 0)
    def _init():
        acc_ref[...] = jnp.zeros_like(acc_ref)

    dcol = dinv8_ref[:, 0:1]
    S = sp_ref[0] + sp_ref[1] + p_ref[...]
    h1 = jnp.maximum(dcol * S + b1_ref[...], 0.0)
    t = jnp.sum(tp_ref[...], axis=0)[:, None] + dcol
    row = i * blk + lax.broadcasted_iota(jnp.int32, (blk, 1), 0)
    cw = jnp.where(row < N, dcol * t, 0.0)
    acc_ref[...] += jnp.sum(cw * h1, axis=0, keepdims=True)

    @pl.when(i == pl.num_programs(0) - 1)
    def _fin():
        v = acc_ref[...]
        g = jnp.dot(v, w2_ref[...], preferred_element_type=jnp.float32)
        g = g / float(N) + b2_ref[...]
        out_ref[...] = jnp.dot(g, wfc_ref[...],
                               preferred_element_type=jnp.float32) + bfc_ref[...]


def _tc_finish(s_parts, p, dinv8, t_parts, W2, b1, b2, Wfc, bfc):
    blk = 1280
    grid = NPAD // blk
    return pl.pallas_call(
        _tc_finish_body,
        grid=(grid,),
        in_specs=[
            pl.BlockSpec((NC, blk, D), lambda i: (0, i, 0)),
            pl.BlockSpec((blk, D), lambda i: (i, 0)),
            pl.BlockSpec((blk, 8), lambda i: (i, 0)),
            pl.BlockSpec((NW, blk), lambda i: (0, i)),
            pl.BlockSpec((D, D), lambda i: (0, 0)),
            pl.BlockSpec((1, D), lambda i: (0, 0)),
            pl.BlockSpec((1, D), lambda i: (0, 0)),
            pl.BlockSpec((D, 1024), lambda i: (0, 0)),
            pl.BlockSpec((1, 1024), lambda i: (0, 0)),
        ],
        out_specs=pl.BlockSpec((1, 1024), lambda i: (0, 0)),
        out_shape=jax.ShapeDtypeStruct((1, 1024), jnp.float32),
        scratch_shapes=[pltpu.VMEM((1, D), jnp.float32)],
    )(s_parts, p, dinv8, t_parts, W2, b1.reshape(1, D), b2.reshape(1, D),
      Wfc, bfc.reshape(1, 1024))


def kernel(x, edge_index, W1, b1, W2, b2, Wfc, bfc):
    src = edge_index[0].astype(jnp.int32)
    dst = edge_index[1].astype(jnp.int32)
    fill = (jnp.arange(EPAD, dtype=jnp.int32) % (NPAD - N)) + N
    src_p = fill.at[:E].set(src).reshape(NW, NCH, CH)
    dst_p = fill.at[:E].set(dst).reshape(NW, NCH, CH)
    x_pad = jnp.zeros((NPAD, D), jnp.float32).at[:N].set(x)

    z1 = jnp.zeros((NPAD,), jnp.float32)
    z2d = jnp.zeros((RPW, D), jnp.float32)

    deg_parts = _sc_degree(dst_p, z1)
    p, dinv8 = _tc_prescale(x_pad, W1, deg_parts)
    s_parts = _sc_edge_pass(p, src_p, dst_p, z2d)
    t_parts = _sc_t_pass(dinv8[:, 0], src_p, dst_p, z1)
    out = _tc_finish(s_parts, p, dinv8, t_parts, W2, b1, b2, Wfc, bfc)
    return out.reshape(16, 64)
